# baseline (speedup 1.0000x reference)
import functools

import jax
import jax.numpy as jnp
from jax import lax
from jax.experimental import pallas as pl
from jax.experimental.pallas import tpu as pltpu
from jax.experimental.pallas import tpu_sc as plsc

_NUM_CORES = 2
_NUM_SUBCORES = 16
_NW = _NUM_CORES * _NUM_SUBCORES

_CHUNK = 256
_NBUF = 10


def _make_sc_gather(n_chunks: int, d: int):
    mesh = plsc.VectorSubcoreMesh(core_axis_name="c", subcore_axis_name="s")

    @functools.partial(
        pl.kernel,
        mesh=mesh,
        out_type=jax.ShapeDtypeStruct((_NW, n_chunks, _CHUNK, d), jnp.float32),
        compiler_params=pltpu.CompilerParams(use_tc_tiling_on_sc=False),
        scratch_types=(
            [pltpu.VMEM((n_chunks, _CHUNK), jnp.int32)]
            + [pltpu.VMEM((_CHUNK, d), jnp.float32) for _ in range(_NBUF)]
            + [pltpu.SemaphoreType.DMA for _ in range(2 * _NBUF)]
        ),
    )
    def gather_kernel(table_hbm, idx_hbm, out_hbm, idx_v, *bufs_and_sems):
        bufs = bufs_and_sems[:_NBUF]
        gsem = bufs_and_sems[_NBUF:2 * _NBUF]
        ssem = bufs_and_sems[2 * _NBUF:]
        wid = lax.axis_index("s") * _NUM_CORES + lax.axis_index("c")

        pltpu.sync_copy(idx_hbm.at[wid], idx_v)

        def gather_start(chunk, b):
            pltpu.make_async_copy(
                table_hbm.at[idx_v.at[chunk]], bufs[b], gsem[b]
            ).start()

        def gather_wait(chunk, b):
            pltpu.make_async_copy(
                table_hbm.at[idx_v.at[chunk]], bufs[b], gsem[b]
            ).wait()

        def store_start(chunk, b):
            pltpu.make_async_copy(
                bufs[b], out_hbm.at[wid, chunk], ssem[b]
            ).start()

        def store_wait(chunk, b):
            pltpu.make_async_copy(
                bufs[b], out_hbm.at[wid, chunk], ssem[b]
            ).wait()

        for b in range(_NBUF):
            gather_start(b, b)

        n_groups = n_chunks // _NBUF

        def group_body(g, carry):
            for b in range(_NBUF):
                c = g * _NBUF + b
                gather_wait(c, b)
                store_start(c, b)
            for b in range(_NBUF):
                c = g * _NBUF + b
                nc = c + _NBUF

                @pl.when(nc < n_chunks)
                def _():
                    store_wait(c, b)
                    gather_start(nc, b)

            return carry

        lax.fori_loop(0, n_groups, group_body, 0)

        for b in range(_NBUF):
            c = (n_groups - 1) * _NBUF + b
            store_wait(c, b)

    return gather_kernel


def kernel(x, table):
    batch, hist = x.shape
    vocab, d = table.shape
    total = batch * hist
    assert total % (_NW * _CHUNK) == 0
    n_chunks = total // (_NW * _CHUNK)
    assert n_chunks % _NBUF == 0

    idx = x.reshape(_NW, n_chunks, _CHUNK)
    out = _make_sc_gather(n_chunks, d)(table, idx)
    return out.reshape(batch, hist, d)


# speedup vs baseline: 1.1406x; 1.1406x over previous
import functools

import jax
import jax.numpy as jnp
from jax import lax
from jax.experimental import pallas as pl
from jax.experimental.pallas import tpu as pltpu
from jax.experimental.pallas import tpu_sc as plsc

_NUM_CORES = 2
_NUM_SUBCORES = 16
_NW = _NUM_CORES * _NUM_SUBCORES

_L = 16
_BT = 128
_NBUF = 4


def _make_sc_gather(hist: int, d: int, batch: int):
    assert batch == _NW * _NBUF * _BT
    nd8 = d // 8
    mesh = plsc.VectorSubcoreMesh(core_axis_name="c", subcore_axis_name="s")

    @functools.partial(
        pl.kernel,
        mesh=mesh,
        out_type=jax.ShapeDtypeStruct((hist, nd8, batch // _BT, 8, _BT),
                                      jnp.float32),
        compiler_params=pltpu.CompilerParams(
            use_tc_tiling_on_sc=False, needs_layout_passes=False,
        ),
        scratch_types=(
            [pltpu.VMEM((hist, _NBUF * _BT), jnp.int32)]
            + [pltpu.VMEM((_BT, d), jnp.float32) for _ in range(_NBUF)]
            + [pltpu.VMEM((nd8, 8, _BT), jnp.float32) for _ in range(_NBUF)]
            + [pltpu.SemaphoreType.DMA for _ in range(2 * _NBUF)]
        ),
    )
    def gather_kernel(table_hbm, xt_hbm, out_hbm, idx_v, *rest):
        bufs = rest[:_NBUF]
        tbufs = rest[_NBUF:2 * _NBUF]
        gsem = rest[2 * _NBUF:3 * _NBUF]
        ssem = rest[3 * _NBUF:]
        wid = lax.axis_index("s") * _NUM_CORES + lax.axis_index("c")

        pltpu.sync_copy(xt_hbm.at[:, pl.ds(wid * (_NBUF * _BT), _NBUF * _BT)],
                        idx_v)

        def gather_start(j, b):
            pltpu.make_async_copy(
                table_hbm.at[idx_v.at[j, pl.ds(b * _BT, _BT)]],
                bufs[b], gsem[b],
            ).start()

        def gather_wait(j, b):
            pltpu.make_async_copy(
                table_hbm.at[idx_v.at[j, pl.ds(b * _BT, _BT)]],
                bufs[b], gsem[b],
            ).wait()

        def store_start(j, b):
            pltpu.make_async_copy(
                tbufs[b], out_hbm.at[j, :, wid * _NBUF + b], ssem[b],
            ).start()

        def store_wait(j, b):
            pltpu.make_async_copy(
                tbufs[b], out_hbm.at[j, :, wid * _NBUF + b], ssem[b],
            ).wait()

        ridx = [jnp.arange(_L, dtype=jnp.int32) + k * _L
                for k in range(_BT // _L)]

        def transpose_block(b):
            buf, tbuf = bufs[b], tbufs[b]
            for dt in range(nd8):
                for ds_ in range(8):
                    dcol = jnp.full((_L,), dt * 8 + ds_, jnp.int32)
                    for k in range(_BT // _L):
                        v = plsc.load_gather(buf, [ridx[k], dcol])
                        tbuf[dt, ds_, pl.ds(k * _L, _L)] = v

        for b in range(_NBUF):
            gather_start(0, b)

        def j_body(j, carry):
            for b in range(_NBUF):
                gather_wait(j, b)

                @pl.when(j > 0)
                def _():
                    store_wait(j - 1, b)

                transpose_block(b)
                store_start(j, b)

                @pl.when(j < hist - 1)
                def _():
                    gather_start(j + 1, b)

            return carry

        lax.fori_loop(0, hist, j_body, 0)

        for b in range(_NBUF):
            store_wait(hist - 1, b)

    return gather_kernel


def kernel(x, table):
    batch, hist = x.shape
    vocab, d = table.shape
    out5 = _make_sc_gather(hist, d, batch)(table, x.T)
    return out5.transpose(2, 4, 0, 1, 3).reshape(batch, hist, d)


# speedup vs baseline: 1.6753x; 1.4688x over previous
import functools

import jax
import jax.numpy as jnp
from jax import lax
from jax.experimental import pallas as pl
from jax.experimental.pallas import tpu as pltpu
from jax.experimental.pallas import tpu_sc as plsc

_NUM_CORES = 2
_NUM_SUBCORES = 16
_NW = _NUM_CORES * _NUM_SUBCORES

_L = 16
_BT = 128
_NBUF = 4


def _make_sc_gather(hist: int, d: int, batch: int):
    assert batch == _NW * _NBUF * _BT
    nd8 = d // 8
    mesh = plsc.VectorSubcoreMesh(core_axis_name="c", subcore_axis_name="s")

    @functools.partial(
        pl.kernel,
        mesh=mesh,
        out_type=jax.ShapeDtypeStruct((hist, nd8, batch // _BT, 8, _BT),
                                      jnp.float32),
        compiler_params=pltpu.CompilerParams(
            use_tc_tiling_on_sc=False, needs_layout_passes=False,
        ),
        scratch_types=(
            [pltpu.VMEM((hist, _NBUF * _BT), jnp.int32)]
            + [pltpu.VMEM((_BT, d), jnp.float32) for _ in range(_NBUF)]
            + [pltpu.VMEM((nd8, 8, _BT), jnp.float32) for _ in range(_NBUF)]
            + [pltpu.SemaphoreType.DMA for _ in range(2 * _NBUF)]
        ),
    )
    def gather_kernel(table_hbm, xt_hbm, out_hbm, idx_v, *rest):
        bufs = rest[:_NBUF]
        tbufs = rest[_NBUF:2 * _NBUF]
        gsem = rest[2 * _NBUF:3 * _NBUF]
        ssem = rest[3 * _NBUF:]
        wid = lax.axis_index("s") * _NUM_CORES + lax.axis_index("c")

        pltpu.sync_copy(xt_hbm.at[:, pl.ds(wid * (_NBUF * _BT), _NBUF * _BT)],
                        idx_v)

        def gather_start(j, b):
            pltpu.make_async_copy(
                table_hbm.at[idx_v.at[j, pl.ds(b * _BT, _BT)]],
                bufs[b], gsem[b],
            ).start()

        def gather_wait(j, b):
            pltpu.make_async_copy(
                table_hbm.at[idx_v.at[j, pl.ds(b * _BT, _BT)]],
                bufs[b], gsem[b],
            ).wait()

        def store_start(j, b):
            pltpu.make_async_copy(
                tbufs[b], out_hbm.at[j, :, wid * _NBUF + b], ssem[b],
            ).start()

        def store_wait(j, b):
            pltpu.make_async_copy(
                tbufs[b], out_hbm.at[j, :, wid * _NBUF + b], ssem[b],
            ).wait()

        ridx = [jnp.arange(_L, dtype=jnp.int32) + k * _L
                for k in range(_BT // _L)]

        def transpose_block(b):
            buf, tbuf = bufs[b], tbufs[b]

            @plsc.parallel_loop(0, d, unroll=8)
            def _(dd):
                dcol = jnp.full((_L,), dd, jnp.int32)
                dt = dd // 8
                ds_ = dd % 8
                for k in range(_BT // _L):
                    v = plsc.load_gather(buf, [ridx[k], dcol])
                    tbuf[dt, ds_, pl.ds(k * _L, _L)] = v

        for b in range(_NBUF):
            gather_start(0, b)

        def j_body(j, carry):
            for b in range(_NBUF):
                gather_wait(j, b)

                @pl.when(j > 0)
                def _():
                    store_wait(j - 1, b)

                transpose_block(b)
                store_start(j, b)

                @pl.when(j < hist - 1)
                def _():
                    gather_start(j + 1, b)

            return carry

        lax.fori_loop(0, hist, j_body, 0)

        for b in range(_NBUF):
            store_wait(hist - 1, b)

    return gather_kernel


def kernel(x, table):
    batch, hist = x.shape
    vocab, d = table.shape
    out5 = _make_sc_gather(hist, d, batch)(table, x.T)
    return out5.transpose(2, 4, 0, 1, 3).reshape(batch, hist, d)


# speedup vs baseline: 2.4455x; 1.4597x over previous
import functools

import jax
import jax.numpy as jnp
from jax import lax
from jax.experimental import pallas as pl
from jax.experimental.pallas import tpu as pltpu
from jax.experimental.pallas import tpu_sc as plsc

_NUM_CORES = 2
_NUM_SUBCORES = 16
_NW = _NUM_CORES * _NUM_SUBCORES

_L = 16
_BT = 128
_NBUF = 4


def _make_sc_gather(hist: int, d: int, batch: int):
    assert batch == _NW * _NBUF * _BT
    nd8 = d // 8
    mesh = plsc.VectorSubcoreMesh(core_axis_name="c", subcore_axis_name="s")

    @functools.partial(
        pl.kernel,
        mesh=mesh,
        out_type=jax.ShapeDtypeStruct((hist, nd8, batch // _BT, 8, _BT),
                                      jnp.float32),
        compiler_params=pltpu.CompilerParams(
            use_tc_tiling_on_sc=False, needs_layout_passes=False,
        ),
        scratch_types=(
            [pltpu.VMEM((hist, _NBUF * _BT), jnp.int32)]
            + [pltpu.VMEM((_BT, d), jnp.float32) for _ in range(_NBUF)]
            + [pltpu.VMEM((nd8, 8, _BT + 1), jnp.float32) for _ in range(_NBUF)]
            + [pltpu.SemaphoreType.DMA for _ in range(2 * _NBUF)]
        ),
    )
    def gather_kernel(table_hbm, xt_hbm, out_hbm, idx_v, *rest):
        bufs = rest[:_NBUF]
        tbufs = rest[_NBUF:2 * _NBUF]
        gsem = rest[2 * _NBUF:3 * _NBUF]
        ssem = rest[3 * _NBUF:]
        wid = lax.axis_index("s") * _NUM_CORES + lax.axis_index("c")

        pltpu.sync_copy(xt_hbm.at[:, pl.ds(wid * (_NBUF * _BT), _NBUF * _BT)],
                        idx_v)

        def gather_start(j, b):
            pltpu.make_async_copy(
                table_hbm.at[idx_v.at[j, pl.ds(b * _BT, _BT)]],
                bufs[b], gsem[b],
            ).start()

        def gather_wait(j, b):
            pltpu.make_async_copy(
                table_hbm.at[idx_v.at[j, pl.ds(b * _BT, _BT)]],
                bufs[b], gsem[b],
            ).wait()

        def store_start(j, b):
            pltpu.make_async_copy(
                tbufs[b].at[:, :, pl.ds(0, _BT)],
                out_hbm.at[j, :, wid * _NBUF + b], ssem[b],
            ).start()

        def store_wait(j, b):
            pltpu.make_async_copy(
                tbufs[b].at[:, :, pl.ds(0, _BT)],
                out_hbm.at[j, :, wid * _NBUF + b], ssem[b],
            ).wait()

        lane = jnp.arange(_L, dtype=jnp.int32)
        dt_idx = [lane // 8 + (c * _L) // 8 for c in range(d // _L)]
        ds_idx = lane % 8

        def transpose_block(b):
            buf, tbuf = bufs[b], tbufs[b]

            @plsc.parallel_loop(0, _BT, unroll=8)
            def _(r):
                rvec = jnp.full((_L,), r, jnp.int32)
                for c in range(d // _L):
                    v = buf[r, pl.ds(c * _L, _L)]
                    plsc.store_scatter(tbuf, [dt_idx[c], ds_idx, rvec], v)

        for b in range(_NBUF):
            gather_start(0, b)

        def j_body(j, carry):
            for b in range(_NBUF):
                gather_wait(j, b)

                @pl.when(j > 0)
                def _():
                    store_wait(j - 1, b)

                transpose_block(b)
                store_start(j, b)

                @pl.when(j < hist - 1)
                def _():
                    gather_start(j + 1, b)

            return carry

        lax.fori_loop(0, hist, j_body, 0)

        for b in range(_NBUF):
            store_wait(hist - 1, b)

    return gather_kernel


def kernel(x, table):
    batch, hist = x.shape
    vocab, d = table.shape
    out5 = _make_sc_gather(hist, d, batch)(table, x.T)
    return out5.transpose(2, 4, 0, 1, 3).reshape(batch, hist, d)
